# baseline (speedup 1.0000x reference)
import jax
import jax.numpy as jnp
from jax.experimental import pallas as pl
from jax.experimental.pallas import tpu as pltpu
from jax.experimental.pallas import tpu_sc as plsc

_MAX_SEQLEN = 4096
_SEG_W = 4096
_TQ = 512
_TK = 2048
_NSHIFT = 8


def _tc_add_kernel(seg_ref, attn_ref, out_ref, u_ref):
    i = pl.program_id(1)

    @pl.when(i == 0)
    def _build_u():
        seg = seg_ref[0, 0, :]
        v = jnp.broadcast_to(seg[None, :], (8, _SEG_W))
        row = jax.lax.broadcasted_iota(jnp.int32, (8, _SEG_W), 0)
        for t in range(3):
            n = 1 << t
            rolled = pltpu.roll(v, _SEG_W - n, axis=1)
            v = jnp.where((row & n) == 0, rolled, v)
        for a in range(_TQ // 8):
            n = _TQ - 8 - 8 * a
            slab = pltpu.roll(v, _SEG_W - n, axis=1) if n else v
            u_ref[8 * a:8 * (a + 1), :] = slab

    for ii in range(2048 // _TQ):
        @pl.when(i == ii)
        def _consume(ii=ii):
            x0 = (_MAX_SEQLEN // 2 - _TQ) - _TQ * ii
            out_ref[0, 0] = attn_ref[0, 0] + u_ref[:, x0:x0 + _TK]


def _tc_out(attn_mtx, seg):
    b, h, q, k = attn_mtx.shape
    blk = pl.BlockSpec((1, 1, _TQ, _TK), lambda hh, ii: (0, hh, ii, 0))
    return pl.pallas_call(
        _tc_add_kernel,
        grid=(h, q // _TQ),
        in_specs=[
            pl.BlockSpec((1, 1, _SEG_W), lambda hh, ii: (hh, 0, 0)),
            blk,
        ],
        out_specs=blk,
        out_shape=jax.ShapeDtypeStruct((b, h, q, k), jnp.float32),
        scratch_shapes=[pltpu.VMEM((_TQ, _SEG_W), jnp.float32)],
        compiler_params=pltpu.CompilerParams(
            dimension_semantics=("parallel", "arbitrary"),
        ),
    )(seg, attn_mtx)


def _sc_bias(seg_shift_flat, nh, nq, nk):
    mesh = plsc.VectorSubcoreMesh(core_axis_name="c", subcore_axis_name="s")
    rows_per_unit = nh * nq // 32
    ngroups = rows_per_unit // _NSHIFT

    @pl.kernel(
        out_type=jax.ShapeDtypeStruct((nh * nq * nk,), jnp.float32),
        mesh=mesh,
        scratch_types=(
            [pltpu.VMEM((_SEG_W,), jnp.float32) for _ in range(_NSHIFT)]
            + [pltpu.SemaphoreType.DMA]
        ),
    )
    def body(seg_hbm, bias_hbm, *scr):
        segs, sem = scr[:_NSHIFT], scr[_NSHIFT]
        c = jax.lax.axis_index("c")
        s = jax.lax.axis_index("s")
        u = c * 16 + s
        h = u // 2
        q0 = (u % 2) * rows_per_unit
        for j in range(_NSHIFT):
            pltpu.sync_copy(
                seg_hbm.at[pl.ds((j * nh + h) * _SEG_W, _SEG_W)], segs[j]
            )

        row0 = (h * nq + q0) * nk

        def group(g, wait):
            base = pl.multiple_of((nq - _NSHIFT) - q0 - _NSHIFT * g, _NSHIFT)
            dst0 = pl.multiple_of(row0 + _NSHIFT * g * nk, _NSHIFT)
            for e in range(_NSHIFT):
                pltpu.make_async_copy(
                    segs[_NSHIFT - 1 - e].at[pl.ds(base, nk)],
                    bias_hbm.at[pl.ds(dst0 + e * nk, nk)],
                    sem,
                ).start()
            if wait:
                for _ in range(_NSHIFT):
                    pltpu.make_async_copy(
                        segs[0].at[pl.ds(0, nk)],
                        bias_hbm.at[pl.ds(row0, nk)],
                        sem,
                    ).wait()

        prime_groups = 8
        for g0 in range(prime_groups):
            group(g0, False)

        @pl.loop(prime_groups, ngroups)
        def _steady(g):
            group(g, True)

        for _ in range(prime_groups * _NSHIFT):
            pltpu.make_async_copy(
                segs[0].at[pl.ds(0, nk)],
                bias_hbm.at[pl.ds(row0, nk)],
                sem,
            ).wait()

    return body(seg_shift_flat)


def kernel(attn_mtx, embedding_table):
    b, h, q, k = attn_mtx.shape
    assert (b, h, q, k) == (1, 16, 2048, 2048)
    seg2d = jnp.flip(embedding_table[2048:6143, :], axis=0).T
    segp = jnp.pad(seg2d, ((0, 0), (0, _SEG_W + _NSHIFT - seg2d.shape[1])))
    seg_shift = jnp.stack(
        [segp[:, j:j + _SEG_W] for j in range(_NSHIFT)], axis=0
    )
    seg3d = segp[:, None, :_SEG_W]

    bias = _sc_bias(seg_shift.reshape(-1), h, q, k).reshape(b, h, q, k)
    out = _tc_out(attn_mtx, seg3d)
    return out, bias


# speedup vs baseline: 1.0147x; 1.0147x over previous
import jax
import jax.numpy as jnp
from jax.experimental import pallas as pl
from jax.experimental.pallas import tpu as pltpu
from jax.experimental.pallas import tpu_sc as plsc

_MAX_SEQLEN = 4096
_SEG_W = 4096
_TQ = 512
_TK = 2048
_NSHIFT = 8


def _tc_add_kernel(seg_ref, attn_ref, out_ref, u_ref):
    i = pl.program_id(1)

    @pl.when(i == 0)
    def _build_u():
        seg = seg_ref[0, 0, :]
        v = jnp.broadcast_to(seg[None, :], (8, _SEG_W))
        row = jax.lax.broadcasted_iota(jnp.int32, (8, _SEG_W), 0)
        for t in range(3):
            n = 1 << t
            rolled = pltpu.roll(v, _SEG_W - n, axis=1)
            v = jnp.where((row & n) == 0, rolled, v)
        for a in range(_TQ // 8):
            n = _TQ - 8 - 8 * a
            slab = pltpu.roll(v, _SEG_W - n, axis=1) if n else v
            u_ref[8 * a:8 * (a + 1), :] = slab

    for ii in range(2048 // _TQ):
        @pl.when(i == ii)
        def _consume(ii=ii):
            x0 = (_MAX_SEQLEN // 2 - _TQ) - _TQ * ii
            out_ref[0, 0] = attn_ref[0, 0] + u_ref[:, x0:x0 + _TK]


def _tc_out(attn_mtx, seg):
    b, h, q, k = attn_mtx.shape
    blk = pl.BlockSpec((1, 1, _TQ, _TK), lambda hh, ii: (0, hh, ii, 0))
    return pl.pallas_call(
        _tc_add_kernel,
        grid=(h, q // _TQ),
        in_specs=[
            pl.BlockSpec((1, 1, _SEG_W), lambda hh, ii: (hh, 0, 0)),
            blk,
        ],
        out_specs=blk,
        out_shape=jax.ShapeDtypeStruct((b, h, q, k), jnp.float32),
        scratch_shapes=[pltpu.VMEM((_TQ, _SEG_W), jnp.float32)],
        compiler_params=pltpu.CompilerParams(
            dimension_semantics=("parallel", "arbitrary"),
        ),
    )(seg, attn_mtx)


def _sc_bias(seg_shift, nh, nq, nk):
    mesh = plsc.VectorSubcoreMesh(core_axis_name="c", subcore_axis_name="s")
    heads_per_core = nh // 2
    groups_per_head = nq // 8
    groups_per_tec = groups_per_head // 16

    @pl.kernel(
        out_type=jax.ShapeDtypeStruct((nh * nq, nk), jnp.float32),
        mesh=mesh,
        scratch_types=[
            pltpu.VMEM_SHARED((128, _SEG_W), jnp.float32),
            pltpu.SemaphoreType.DMA,
        ],
    )
    def body(seg_hbm, bias_hbm, s2, sem):
        c = jax.lax.axis_index("c")
        s = jax.lax.axis_index("s")

        @pl.loop(0, heads_per_core)
        def _per_head(hh):
            h = c * heads_per_core + hh
            @pl.when(s == 0)
            def _stage():
                pltpu.sync_copy(seg_hbm.at[h], s2)

            plsc.subcore_barrier()

            def issue(gi):
                g = s * groups_per_tec + gi
                start0 = (nq - 1) - 8 * g
                j0 = start0 & 127
                sub0 = pl.multiple_of(127 - j0, 8)
                base = pl.multiple_of(start0 - j0, 128)
                dst = pl.multiple_of(h * nq + 8 * g, 8)
                pltpu.make_async_copy(
                    s2.at[pl.ds(sub0, 8), pl.ds(base, nk)],
                    bias_hbm.at[pl.ds(dst, 8), :],
                    sem,
                ).start()

            @pl.loop(0, groups_per_tec)
            def _issue_all(gi):
                issue(gi)

            @pl.loop(0, groups_per_tec)
            def _drain(_):
                pltpu.make_async_copy(
                    s2.at[pl.ds(0, 8), pl.ds(0, nk)],
                    bias_hbm.at[pl.ds(0, 8), :],
                    sem,
                ).wait()

            plsc.subcore_barrier()

    return body(seg_shift)


def kernel(attn_mtx, embedding_table):
    b, h, q, k = attn_mtx.shape
    assert (b, h, q, k) == (1, 16, 2048, 2048)
    seg2d = jnp.flip(embedding_table[2048:6143, :], axis=0).T
    segp = jnp.pad(seg2d, ((0, 0), (0, _SEG_W + 128 - seg2d.shape[1])))
    seg_shift = jnp.stack(
        [segp[:, 127 - r:127 - r + _SEG_W] for r in range(128)], axis=1
    )
    seg3d = segp[:, None, :_SEG_W]

    bias = _sc_bias(seg_shift, h, q, k).reshape(b, h, q, k)
    out = _tc_out(attn_mtx, seg3d)
    return out, bias


# speedup vs baseline: 2.0881x; 2.0579x over previous
import jax
import jax.numpy as jnp
from jax.experimental import pallas as pl
from jax.experimental.pallas import tpu as pltpu

_MAX_SEQLEN = 4096
_SEG_W = 4096
_TQ = 512
_TK = 2048


def _build_u(seg_ref, u_ref):
    seg = seg_ref[0, 0, :]
    v = jnp.broadcast_to(seg[None, :], (8, _SEG_W))
    row = jax.lax.broadcasted_iota(jnp.int32, (8, _SEG_W), 0)
    for t in range(3):
        n = 1 << t
        rolled = pltpu.roll(v, _SEG_W - n, axis=1)
        v = jnp.where((row & n) == 0, rolled, v)
    for a in range(_TQ // 8):
        n = _TQ - 8 - 8 * a
        slab = pltpu.roll(v, _SEG_W - n, axis=1) if n else v
        u_ref[8 * a:8 * (a + 1), :] = slab


def _bias_kernel(seg_ref, bias_ref, u_ref):
    i = pl.program_id(1)

    @pl.when(i == 0)
    def _gen():
        _build_u(seg_ref, u_ref)

    for ii in range(2048 // _TQ):
        @pl.when(i == ii)
        def _consume(ii=ii):
            x0 = (_MAX_SEQLEN // 2 - _TQ) - _TQ * ii
            bias_ref[0, 0] = u_ref[:, x0:x0 + _TK]


def _add_kernel(seg_ref, attn_ref, out_ref, u_ref):
    i = pl.program_id(1)

    @pl.when(i == 0)
    def _gen():
        _build_u(seg_ref, u_ref)

    for ii in range(2048 // _TQ):
        @pl.when(i == ii)
        def _consume(ii=ii):
            x0 = (_MAX_SEQLEN // 2 - _TQ) - _TQ * ii
            out_ref[0, 0] = attn_ref[0, 0] + u_ref[:, x0:x0 + _TK]


def kernel(attn_mtx, embedding_table):
    b, h, q, k = attn_mtx.shape
    assert (b, h, q, k) == (1, 16, 2048, 2048)
    seg = jnp.flip(embedding_table[2048:6143, :], axis=0).T
    seg = jnp.pad(seg, ((0, 0), (0, _SEG_W - seg.shape[1])))[:, None, :]

    grid = (h, q // _TQ)
    blk = pl.BlockSpec((1, 1, _TQ, _TK), lambda hh, ii: (0, hh, ii, 0))
    segspec = pl.BlockSpec((1, 1, _SEG_W), lambda hh, ii: (hh, 0, 0))
    cp = pltpu.CompilerParams(dimension_semantics=("parallel", "arbitrary"))
    shp = jax.ShapeDtypeStruct((b, h, q, k), jnp.float32)

    bias = pl.pallas_call(
        _bias_kernel,
        grid=grid,
        in_specs=[segspec],
        out_specs=blk,
        out_shape=shp,
        scratch_shapes=[pltpu.VMEM((_TQ, _SEG_W), jnp.float32)],
        compiler_params=cp,
    )(seg)
    out = pl.pallas_call(
        _add_kernel,
        grid=grid,
        in_specs=[segspec, blk],
        out_specs=blk,
        out_shape=shp,
        scratch_shapes=[pltpu.VMEM((_TQ, _SEG_W), jnp.float32)],
        compiler_params=cp,
    )(seg, attn_mtx)
    return out, bias


# speedup vs baseline: 2.1297x; 1.0199x over previous
import jax
import jax.numpy as jnp
from jax.experimental import pallas as pl
from jax.experimental.pallas import tpu as pltpu

_MAX_SEQLEN = 4096
_SEG_W = 4096
_TQ = 512
_TK = 2048


def _rel_bias_kernel(seg_ref, attn_ref, out_ref, bias_ref, u_ref):
    i = pl.program_id(1)

    @pl.when(i == 0)
    def _build_u():
        seg = seg_ref[0, 0, :]
        v = jnp.broadcast_to(seg[None, :], (8, _SEG_W))
        row = jax.lax.broadcasted_iota(jnp.int32, (8, _SEG_W), 0)
        for t in range(3):
            n = 1 << t
            rolled = pltpu.roll(v, _SEG_W - n, axis=1)
            v = jnp.where((row & n) == 0, rolled, v)
        for a in range(_TQ // 8):
            n = _TQ - 8 - 8 * a
            slab = pltpu.roll(v, _SEG_W - n, axis=1) if n else v
            u_ref[8 * a:8 * (a + 1), :] = slab

    for ii in range(2048 // _TQ):
        @pl.when(i == ii)
        def _consume(ii=ii):
            x0 = (_MAX_SEQLEN // 2 - _TQ) - _TQ * ii
            bias_t = u_ref[:, x0:x0 + _TK]
            out_ref[0, 0] = attn_ref[0, 0] + bias_t
            bias_ref[0, 0] = bias_t


def kernel(attn_mtx, embedding_table):
    b, h, q, k = attn_mtx.shape
    assert (b, h, q, k) == (1, 16, 2048, 2048)
    seg = jnp.flip(embedding_table[2048:6143, :], axis=0).T
    seg = jnp.pad(seg, ((0, 0), (0, _SEG_W - seg.shape[1])))[:, None, :]

    grid = (h, q // _TQ)
    blk = pl.BlockSpec((1, 1, _TQ, _TK), lambda hh, ii: (0, hh, ii, 0))
    out, bias = pl.pallas_call(
        _rel_bias_kernel,
        grid=grid,
        in_specs=[
            pl.BlockSpec((1, 1, _SEG_W), lambda hh, ii: (hh, 0, 0)),
            blk,
        ],
        out_specs=[blk, blk],
        out_shape=[jax.ShapeDtypeStruct((b, h, q, k), jnp.float32)] * 2,
        scratch_shapes=[pltpu.VMEM((_TQ, _SEG_W), jnp.float32)],
        compiler_params=pltpu.CompilerParams(
            dimension_semantics=("parallel", "arbitrary"),
        ),
    )(seg, attn_mtx)
    return out, bias
